# baseline (speedup 1.0000x reference)
import jax
import jax.numpy as jnp
from jax.experimental import pallas as pl
from jax.experimental.pallas import tpu as pltpu

N = 10000
NFEAT = 256
NHID = 128
BM = 400
NPAIR = BM // 2


def _support_kernel(x_ref, w_ref, out_ref):
    out_ref[...] = jnp.dot(x_ref[...], w_ref[...],
                           preferred_element_type=jnp.float32)


def _main_kernel(adj_ref, sup_ref, b_ref, wd2_ref, bdec_ref, label_ref,
                 out_ref):
    i = pl.program_id(0)
    h = jnp.dot(adj_ref[...], sup_ref[...],
                preferred_element_type=jnp.float32)
    h = jnp.maximum(h + b_ref[...], 0.0)
    u = jnp.dot(h, wd2_ref[...], preferred_element_type=jnp.float32)
    row = jax.lax.broadcasted_iota(jnp.int32, (BM, 1), 0)
    w = jnp.where(row % 2 == 0, u[:, 0:1], u[:, 1:2])
    pr = jax.lax.broadcasted_iota(jnp.int32, (NPAIR, BM), 0)
    ci = jax.lax.broadcasted_iota(jnp.int32, (NPAIR, BM), 1)
    S = (ci // 2 == pr).astype(jnp.float32)
    logits = jnp.dot(S, w, preferred_element_type=jnp.float32) + bdec_ref[0]
    lab = label_ref[...]
    t = (jnp.maximum(logits, 0.0) - logits * lab
         + jnp.log1p(jnp.exp(-jnp.abs(logits))))
    part = jnp.sum(t)

    @pl.when(i == 0)
    def _():
        out_ref[0, 0] = 0.0

    out_ref[0, 0] += part


def kernel(x, adj, label, W_enc, b_enc, W_dec, b_dec):
    support = pl.pallas_call(
        _support_kernel,
        out_shape=jax.ShapeDtypeStruct((N, NHID), jnp.float32),
    )(x, W_enc)

    wd2 = W_dec.reshape(2, NHID).T
    b2 = b_enc.reshape(1, NHID)

    grid = N // BM
    total = pl.pallas_call(
        _main_kernel,
        grid=(grid,),
        in_specs=[
            pl.BlockSpec((BM, N), lambda i: (i, 0)),
            pl.BlockSpec((N, NHID), lambda i: (0, 0)),
            pl.BlockSpec((1, NHID), lambda i: (0, 0)),
            pl.BlockSpec((NHID, 2), lambda i: (0, 0)),
            pl.BlockSpec(memory_space=pltpu.SMEM),
            pl.BlockSpec((NPAIR, 1), lambda i: (i, 0)),
        ],
        out_specs=pl.BlockSpec(memory_space=pltpu.SMEM),
        out_shape=jax.ShapeDtypeStruct((1, 1), jnp.float32),
    )(adj, support, b2, wd2, b_dec, label)

    return total[0, 0] / jnp.float32(N // 2)


# speedup vs baseline: 1.0087x; 1.0087x over previous
import jax
import jax.numpy as jnp
from jax.experimental import pallas as pl
from jax.experimental.pallas import tpu as pltpu

N = 10000
NFEAT = 256
NHID = 128
BM = 400
NPAIR = BM // 2


def _support_kernel(x_ref, w_ref, out_ref):
    out_ref[...] = jnp.dot(x_ref[...], w_ref[...],
                           preferred_element_type=jnp.float32
                           ).astype(jnp.bfloat16)


def _main_kernel(adj_ref, sup_ref, b_ref, wd2_ref, bdec_ref, label_ref,
                 out_ref):
    i = pl.program_id(0)
    h = jnp.dot(adj_ref[...].astype(jnp.bfloat16), sup_ref[...],
                preferred_element_type=jnp.float32)
    h = jnp.maximum(h + b_ref[...], 0.0)
    u = jnp.dot(h, wd2_ref[...], preferred_element_type=jnp.float32)
    row = jax.lax.broadcasted_iota(jnp.int32, (BM, 1), 0)
    w = jnp.where(row % 2 == 0, u[:, 0:1], u[:, 1:2])
    pr = jax.lax.broadcasted_iota(jnp.int32, (NPAIR, BM), 0)
    ci = jax.lax.broadcasted_iota(jnp.int32, (NPAIR, BM), 1)
    S = (ci // 2 == pr).astype(jnp.float32)
    logits = jnp.dot(S, w, preferred_element_type=jnp.float32) + bdec_ref[0]
    lab = label_ref[...]
    t = (jnp.maximum(logits, 0.0) - logits * lab
         + jnp.log1p(jnp.exp(-jnp.abs(logits))))
    part = jnp.sum(t)

    @pl.when(i == 0)
    def _():
        out_ref[0, 0] = 0.0

    out_ref[0, 0] += part


def kernel(x, adj, label, W_enc, b_enc, W_dec, b_dec):
    support = pl.pallas_call(
        _support_kernel,
        out_shape=jax.ShapeDtypeStruct((N, NHID), jnp.bfloat16),
    )(x, W_enc)

    wd2 = W_dec.reshape(2, NHID).T
    b2 = b_enc.reshape(1, NHID)

    grid = N // BM
    total = pl.pallas_call(
        _main_kernel,
        grid=(grid,),
        in_specs=[
            pl.BlockSpec((BM, N), lambda i: (i, 0)),
            pl.BlockSpec((N, NHID), lambda i: (0, 0)),
            pl.BlockSpec((1, NHID), lambda i: (0, 0)),
            pl.BlockSpec((NHID, 2), lambda i: (0, 0)),
            pl.BlockSpec(memory_space=pltpu.SMEM),
            pl.BlockSpec((NPAIR, 1), lambda i: (i, 0)),
        ],
        out_specs=pl.BlockSpec(memory_space=pltpu.SMEM),
        out_shape=jax.ShapeDtypeStruct((1, 1), jnp.float32),
    )(adj, support, b2, wd2, b_dec, label)

    return total[0, 0] / jnp.float32(N // 2)


# speedup vs baseline: 1.0363x; 1.0274x over previous
import jax
import jax.numpy as jnp
from jax.experimental import pallas as pl
from jax.experimental.pallas import tpu as pltpu

N = 10000
NFEAT = 256
NHID = 128
BM = 400
NPAIR = BM // 2
G = N // BM


def _main_kernel(x_ref, we_ref, adj_ref, b_ref, wd2_ref, bdec_ref,
                 lab_ref, out_ref, sup_ref):
    i = pl.program_id(0)

    @pl.when(i == 0)
    def _():
        sup_ref[...] = jnp.dot(x_ref[...].astype(jnp.bfloat16),
                               we_ref[...].astype(jnp.bfloat16),
                               preferred_element_type=jnp.float32
                               ).astype(jnp.bfloat16)
        out_ref[0, 0] = 0.0

    h = jnp.dot(adj_ref[...].astype(jnp.bfloat16), sup_ref[...],
                preferred_element_type=jnp.float32)
    h = jnp.maximum(h + b_ref[...], 0.0)
    u = jnp.dot(h, wd2_ref[...], preferred_element_type=jnp.float32)
    row = jax.lax.broadcasted_iota(jnp.int32, (BM, 1), 0)
    w = jnp.where(row % 2 == 0, u[:, 0:1], u[:, 1:2])
    pr = jax.lax.broadcasted_iota(jnp.int32, (NPAIR, BM), 0)
    ci = jax.lax.broadcasted_iota(jnp.int32, (NPAIR, BM), 1)
    S = (ci // 2 == pr).astype(jnp.float32)
    logits = jnp.dot(S, w, preferred_element_type=jnp.float32) + bdec_ref[0]
    lab = lab_ref[0]
    pos = jnp.sum(jnp.maximum(logits, 0.0)
                  + jnp.log1p(jnp.exp(-jnp.abs(logits))))
    cross = jnp.dot(lab, logits, preferred_element_type=jnp.float32)[0, 0]
    out_ref[0, 0] += pos - cross


def kernel(x, adj, label, W_enc, b_enc, W_dec, b_dec):
    wd2 = W_dec.reshape(2, NHID).T
    b2 = b_enc.reshape(1, NHID)
    lab3 = label.reshape(G, 1, NPAIR)

    total = pl.pallas_call(
        _main_kernel,
        grid=(G,),
        in_specs=[
            pl.BlockSpec((N, NFEAT), lambda i: (0, 0)),
            pl.BlockSpec((NFEAT, NHID), lambda i: (0, 0)),
            pl.BlockSpec((BM, N), lambda i: (i, 0)),
            pl.BlockSpec((1, NHID), lambda i: (0, 0)),
            pl.BlockSpec((NHID, 2), lambda i: (0, 0)),
            pl.BlockSpec(memory_space=pltpu.SMEM),
            pl.BlockSpec((1, 1, NPAIR), lambda i: (i, 0, 0)),
        ],
        out_specs=pl.BlockSpec(memory_space=pltpu.SMEM),
        out_shape=jax.ShapeDtypeStruct((1, 1), jnp.float32),
        scratch_shapes=[pltpu.VMEM((N, NHID), jnp.bfloat16)],
    )(x, W_enc, adj, b2, wd2, b_dec, lab3)

    return total[0, 0] / jnp.float32(N // 2)
